# baseline (speedup 1.0000x reference)
import functools

import jax
import jax.numpy as jnp
from jax import lax
from jax.experimental import pallas as pl
from jax.experimental.pallas import tpu as pltpu

EPS = 1e-05
H = 256


def _edge_mlp_body(g_ref, ea_ref, w1c_ref, w2_ref, b1_ref, b2_ref, m_ref):
    z = g_ref[...] + jnp.dot(ea_ref[...], w1c_ref[...],
                             preferred_element_type=jnp.float32)
    z = jnp.maximum(z + b1_ref[...], 0.0)
    m_ref[...] = jnp.dot(z, w2_ref[...],
                         preferred_element_type=jnp.float32) + b2_ref[...]


def _edge_mlp(g, ea, w1c, w2, b1, b2, block_e):
    e = g.shape[0]
    grid = (e // block_e,)
    return pl.pallas_call(
        _edge_mlp_body,
        grid=grid,
        in_specs=[
            pl.BlockSpec((block_e, H), lambda i: (i, 0)),
            pl.BlockSpec((block_e, H), lambda i: (i, 0)),
            pl.BlockSpec((H, H), lambda i: (0, 0)),
            pl.BlockSpec((H, H), lambda i: (0, 0)),
            pl.BlockSpec((1, H), lambda i: (0, 0)),
            pl.BlockSpec((1, H), lambda i: (0, 0)),
        ],
        out_specs=pl.BlockSpec((block_e, H), lambda i: (i, 0)),
        out_shape=jax.ShapeDtypeStruct((e, H), jnp.float32),
    )(g, ea, w1c, w2, b1, b2)


def _post_body(h_ref, s1_ref, mx_ref, mn_ref, sq_ref, dpk_ref,
               pw1_ref, pb1_ref, pw2_ref, pb2_ref, wab_ref,
               h_out_ref, hds_out_ref):
    h = h_ref[...]
    dc = dpk_ref[:, 0:1]
    logd = dpk_ref[:, 1:2]
    inv_logd = dpk_ref[:, 2:3]
    has_e = dpk_ref[:, 3:4]

    s1 = s1_ref[...]
    mean = s1 / dc
    mx = jnp.where(has_e > 0.0, mx_ref[...], 0.0)
    mn = jnp.where(has_e > 0.0, mn_ref[...], 0.0)
    mean_sq = sq_ref[...] / dc
    std = jnp.sqrt(jnp.maximum(mean_sq - mean * mean, 0.0) + EPS)

    acc = jnp.dot(h, pw1_ref[0], preferred_element_type=jnp.float32)
    aggs = (mean, mx, mn, std)
    for a in range(4):
        agg = aggs[a]
        acc += jnp.dot(agg, pw1_ref[1 + 3 * a],
                       preferred_element_type=jnp.float32)
        acc += jnp.dot(agg * logd, pw1_ref[2 + 3 * a],
                       preferred_element_type=jnp.float32)
        acc += jnp.dot(agg * inv_logd, pw1_ref[3 + 3 * a],
                       preferred_element_type=jnp.float32)
    z = jnp.maximum(acc + pb1_ref[...], 0.0)
    h_new = jnp.dot(z, pw2_ref[...],
                    preferred_element_type=jnp.float32) + pb2_ref[...] + h
    h_out_ref[...] = h_new
    hds_out_ref[...] = jnp.dot(h_new, wab_ref[...],
                               preferred_element_type=jnp.float32)


def _post_mlp(h, s1, mx, mn, sq, dpk, pw1, pb1, pw2, pb2, wab, block_n):
    n = h.shape[0]
    grid = (n // block_n,)
    blk = lambda: pl.BlockSpec((block_n, H), lambda i: (i, 0))
    return pl.pallas_call(
        _post_body,
        grid=grid,
        in_specs=[
            blk(), blk(), blk(), blk(), blk(),
            pl.BlockSpec((block_n, 128), lambda i: (i, 0)),
            pl.BlockSpec((13, H, H), lambda i: (0, 0, 0)),
            pl.BlockSpec((1, H), lambda i: (0, 0)),
            pl.BlockSpec((H, H), lambda i: (0, 0)),
            pl.BlockSpec((1, H), lambda i: (0, 0)),
            pl.BlockSpec((H, 2 * H), lambda i: (0, 0)),
        ],
        out_specs=[
            pl.BlockSpec((block_n, H), lambda i: (i, 0)),
            pl.BlockSpec((block_n, 2 * H), lambda i: (i, 0)),
        ],
        out_shape=[
            jax.ShapeDtypeStruct((n, H), jnp.float32),
            jax.ShapeDtypeStruct((n, 2 * H), jnp.float32),
        ],
    )(h, s1, mx, mn, sq, dpk, pw1, pb1, pw2, pb2, wab)


def _matmul_body(x_ref, w_ref, o_ref):
    o_ref[...] = jnp.dot(x_ref[...], w_ref[...],
                         preferred_element_type=jnp.float32)


def _matmul(x, w, block_m):
    m, k = x.shape
    kk, nn = w.shape
    return pl.pallas_call(
        _matmul_body,
        grid=(m // block_m,),
        in_specs=[
            pl.BlockSpec((block_m, k), lambda i: (i, 0)),
            pl.BlockSpec((kk, nn), lambda i: (0, 0)),
        ],
        out_specs=pl.BlockSpec((block_m, nn), lambda i: (i, 0)),
        out_shape=jax.ShapeDtypeStruct((m, nn), jnp.float32),
    )(x, w)


def kernel(atom_emb, bond_emb, pre_W1, pre_b1, pre_W2, pre_b2,
           post_W1, post_b1, post_W2, post_b2, x, edge_index, edge_attr):
    n = x.shape[0]
    e = edge_index.shape[1]
    l_total = pre_W1.shape[0]

    h = jnp.zeros((n, H), jnp.float32)
    for f in range(atom_emb.shape[0]):
        h = h + jnp.take(atom_emb[f], x[:, f], axis=0)
    ea = jnp.zeros((e, H), jnp.float32)
    for f in range(bond_emb.shape[0]):
        ea = ea + jnp.take(bond_emb[f], edge_attr[:, f], axis=0)

    src = edge_index[0]
    dst = edge_index[1]

    ones = jnp.ones((e,), jnp.float32)
    deg = jax.ops.segment_sum(ones, dst, num_segments=n)
    dc = jnp.maximum(deg, 1.0)
    logd = jnp.log(deg + 1.0)
    inv_logd = 1.0 / jnp.where(logd > 0, logd, 1.0)
    has_e = (deg > 0).astype(jnp.float32)
    dpk = jnp.zeros((n, 128), jnp.float32)
    dpk = dpk.at[:, 0].set(dc).at[:, 1].set(logd)
    dpk = dpk.at[:, 2].set(inv_logd).at[:, 3].set(has_e)

    w_ab = jnp.concatenate([pre_W1[:, 0:H, :], pre_W1[:, H:2 * H, :]],
                           axis=2)
    w_c = pre_W1[:, 2 * H:, :]

    block_n = 1000 if n % 1000 == 0 else n
    block_e = 1280 if e % 1280 == 0 else e
    hds = _matmul(h, w_ab[0], block_n)
    for l in range(l_total):
        hd = jnp.take(hds[:, :H], dst, axis=0)
        hs = jnp.take(hds[:, H:], src, axis=0)
        g = hd + hs
        m = _edge_mlp(g, ea, w_c[l], pre_W2[l], pre_b1[l][None],
                      pre_b2[l][None], block_e)
        s1 = jax.ops.segment_sum(m, dst, num_segments=n)
        mx = jax.ops.segment_max(m, dst, num_segments=n)
        mn = jax.ops.segment_min(m, dst, num_segments=n)
        sq = jax.ops.segment_sum(m * m, dst, num_segments=n)
        wab_next = w_ab[(l + 1) % l_total]
        h, hds = _post_mlp(h, s1, mx, mn, sq, dpk, post_W1[l].reshape(13, H, H),
                           post_b1[l][None], post_W2[l], post_b2[l][None],
                           wab_next, block_n)
    return h


# speedup vs baseline: 3.9644x; 3.9644x over previous
import functools

import jax
import jax.numpy as jnp
from jax import lax
from jax.experimental import pallas as pl
from jax.experimental.pallas import tpu as pltpu
from jax.experimental.pallas import tpu_sc as plsc

EPS = 1e-05
H = 256
NW = 32
GC = 64
RC = 128
NPW = 320
NEG = -3.4e38
POS = 3.4e38


def _sc_mesh():
    return plsc.VectorSubcoreMesh(core_axis_name="c", subcore_axis_name="s")


def _wid():
    return lax.axis_index("s") * 2 + lax.axis_index("c")


def _gather_body(hd, hs, dsti, srci, out,
                 idxd, idxs, rd, rs, semd, sems, semo, *, e_tot):
    w = _wid()
    a0 = ((w * e_tot) // NW) & ~7
    a1 = jnp.where(w == NW - 1, e_tot, (((w + 1) * e_tot) // NW) & ~7)
    nch = (a1 - a0 + GC - 1) // GC

    def pair(g2, carry):
        for b in range(2):
            g = 2 * g2 + b

            @pl.when(g < nch)
            def _():
                c0 = pl.multiple_of(jnp.minimum(a0 + g * GC, a1 - GC), 8)

                @pl.when(g >= 2)
                def _():
                    pltpu.make_async_copy(rd[b], out.at[pl.ds(0, GC)],
                                          semo[b]).wait()

                pltpu.sync_copy(dsti.at[pl.ds(c0, GC)], idxd[b])
                pltpu.sync_copy(srci.at[pl.ds(c0, GC)], idxs[b])
                cpd = pltpu.async_copy(hd.at[idxd[b]], rd[b], semd[b])
                cps = pltpu.async_copy(hs.at[idxs[b]], rs[b], sems[b])
                cpd.wait()
                cps.wait()

                def row(r, _):
                    for c in range(16):
                        sl = pl.ds(c * 16, 16)
                        rd[b][r, sl] = rd[b][r, sl] + rs[b][r, sl]
                    return 0

                lax.fori_loop(0, GC, row, 0)
                pltpu.async_copy(rd[b], out.at[pl.ds(c0, GC)], semo[b])
        return carry

    lax.fori_loop(0, (nch + 1) // 2, pair, 0)

    for b in range(2):
        @pl.when(nch >= b + 1)
        def _():
            pltpu.make_async_copy(rd[b], out.at[pl.ds(0, GC)],
                                  semo[b]).wait()


def _sc_gather(hd, hs, dsti, srci):
    e = dsti.shape[0]
    fn = pl.kernel(
        functools.partial(_gather_body, e_tot=e),
        out_type=jax.ShapeDtypeStruct((e, H), jnp.float32),
        mesh=_sc_mesh(),
        scratch_types=[
            [pltpu.VMEM((GC,), jnp.int32)] * 2,
            [pltpu.VMEM((GC,), jnp.int32)] * 2,
            [pltpu.VMEM((GC, H), jnp.float32)] * 2,
            [pltpu.VMEM((GC, H), jnp.float32)] * 2,
            [pltpu.SemaphoreType.DMA] * 2,
            [pltpu.SemaphoreType.DMA] * 2,
            [pltpu.SemaphoreType.DMA] * 2,
        ],
    )
    return fn(hd, hs, dsti, srci)


def _segred_body(mh, ptrp, out, ptr_v, buf, stage, acc_sav, semi, semo):
    w = _wid()
    n0 = w * NPW
    e_tot = mh.shape[0]
    pltpu.sync_copy(ptrp.at[pl.ds(pl.multiple_of(n0, 8), NPW + 24)], ptr_v)

    def pread(i):
        return ptr_v[pl.ds(i, 16)][0]

    e_lo = pread(0)
    e_hi = pread(NPW)
    e0 = e_lo & ~7
    nch = (e_hi - e0 + RC - 1) // RC
    npairs = (nch + 1) // 2

    zero = jnp.zeros((16,), jnp.float32)
    neg = jnp.full((16,), NEG, jnp.float32)
    pos = jnp.full((16,), POS, jnp.float32)
    init = tuple(([zero] * 8) + ([neg] * 8) + ([pos] * 8) + ([zero] * 8))

    for a4 in range(4):
        for c in range(8):
            acc_sav[a4, pl.ds(c * 16, 16)] = init[8 * a4 + c]

    def flush(i, accs):
        grp = i // 8
        row8 = i % 8

        @pl.when(jnp.logical_and(row8 == 0, grp >= 1))
        def _():
            pltpu.make_async_copy(stage, out.at[:, pl.ds(0, 8), :],
                                  semo).wait()

        for a4 in range(4):
            for c in range(8):
                stage[a4, row8, pl.ds(c * 16, 16)] = accs[8 * a4 + c]

        @pl.when(row8 == 7)
        def _():
            pltpu.async_copy(
                stage,
                out.at[:, pl.ds(pl.multiple_of(n0 + grp * 8, 8), 8), :],
                semo)

    def accum_rows(b, cs_dma, lo, hi, accs):
        def row(e_abs, st):
            r = e_abs - cs_dma
            accs2 = list(st)
            for c in range(8):
                v = buf[b][r, pl.ds(c * 16, 16)]
                accs2[c] = accs2[c] + v
                accs2[8 + c] = jnp.maximum(accs2[8 + c], v)
                accs2[16 + c] = jnp.minimum(accs2[16 + c], v)
                accs2[24 + c] = accs2[24 + c] + v * v
            return tuple(accs2)

        return lax.fori_loop(lo, jnp.maximum(lo, hi), row, accs)

    def run_chunk(g, b, i0):
        cs = e0 + g * RC
        cs_dma = pl.multiple_of(jnp.minimum(cs, e_tot - RC), 8)
        ce = jnp.minimum(cs + RC, e_hi)
        pltpu.make_async_copy(mh.at[pl.ds(0, RC)], buf[b], semi[b]).wait()

        @pl.when(g + 1 < 2 * npairs)
        def _():
            nxt = pl.multiple_of(
                jnp.minimum(e0 + (g + 1) * RC, e_tot - RC), 8)
            pltpu.async_copy(mh.at[pl.ds(nxt, RC)], buf[1 - b], semi[1 - b])

        pn0 = pread(jnp.minimum(i0 + 1, NPW + 7))
        saved = tuple(acc_sav[a4, pl.ds(c * 16, 16)]
                      for a4 in range(4) for c in range(8))
        lo1 = jnp.minimum(jnp.maximum(cs, e_lo), ce)
        accs1 = accum_rows(b, cs_dma, lo1, jnp.minimum(pn0, ce), saved)
        flushed1 = jnp.logical_and(pn0 <= ce, i0 < NPW)

        @pl.when(flushed1)
        def _():
            flush(i0, accs1)

        i1 = i0 + flushed1.astype(jnp.int32)

        def bstep(_, lh):
            lo, hi = lh
            mid = (lo + hi) // 2
            take = pread(jnp.minimum(mid + 1, NPW + 7)) <= ce
            lo2 = jnp.where(jnp.logical_and(lo < hi, take), mid + 1, lo)
            hi2 = jnp.where(jnp.logical_and(lo < hi, jnp.logical_not(take)),
                            mid, hi)
            return lo2, hi2

        i_end, _ = lax.fori_loop(0, 9, bstep, (i1, jnp.int32(NPW)))

        def body2(i, c2):
            accs = accum_rows(b, cs_dma, pread(i), pread(i + 1), init)
            flush(i, accs)
            return c2

        lax.fori_loop(i1, i_end, body2, jnp.int32(0))

        lo3 = jnp.where(flushed1,
                        jnp.minimum(jnp.maximum(
                            pread(jnp.minimum(i_end, NPW + 7)), cs), ce), ce)
        accs3 = accum_rows(b, cs_dma, lo3, ce, init)
        for a4 in range(4):
            for c in range(8):
                k = 8 * a4 + c
                acc_sav[a4, pl.ds(c * 16, 16)] = jnp.where(
                    flushed1, accs3[k], accs1[k])
        return i_end

    def pair(g2, i_c):
        for b in range(2):
            i_c = run_chunk(2 * g2 + b, b, i_c)
        return i_c

    @pl.when(npairs > 0)
    def _():
        pltpu.async_copy(mh.at[pl.ds(pl.multiple_of(
            jnp.minimum(e0, e_tot - RC), 8), RC)], buf[0], semi[0])

    lax.fori_loop(0, npairs, pair, jnp.int32(0))

    pltpu.make_async_copy(stage, out.at[:, pl.ds(0, 8), :], semo).wait()


def _sc_segreduce(mh, ptrp, npad):
    fn = pl.kernel(
        _segred_body,
        out_type=jax.ShapeDtypeStruct((4, npad, 128), jnp.float32),
        mesh=_sc_mesh(),
        scratch_types=[
            pltpu.VMEM((NPW + 24,), jnp.int32),
            [pltpu.VMEM((RC, 128), jnp.float32)] * 2,
            pltpu.VMEM((4, 8, 128), jnp.float32),
            pltpu.VMEM((4, 128), jnp.float32),
            [pltpu.SemaphoreType.DMA] * 2,
            pltpu.SemaphoreType.DMA,
        ],
    )
    return fn(mh, ptrp)


def _encode_body(xp_ref, aemb_ref, wab_ref, h_ref, hd_ref, hs_ref):
    nb = xp_ref.shape[0]
    iota = lax.broadcasted_iota(jnp.int32, (1, 128), 1)
    h = jnp.zeros((nb, H), jnp.float32)
    for f in range(9):
        oh = (xp_ref[:, f][:, None] == iota).astype(jnp.float32)
        h = h + jnp.dot(oh, aemb_ref[f], preferred_element_type=jnp.float32)
    h_ref[...] = h
    hd_ref[...] = jnp.dot(h, wab_ref[0], preferred_element_type=jnp.float32)
    hs_ref[...] = jnp.dot(h, wab_ref[1], preferred_element_type=jnp.float32)


def _encode(xp, aemb, wab, block_n):
    n = xp.shape[0]
    return pl.pallas_call(
        _encode_body,
        grid=(n // block_n,),
        in_specs=[
            pl.BlockSpec((block_n, 16), lambda i: (i, 0)),
            pl.BlockSpec((9, 128, H), lambda i: (0, 0, 0)),
            pl.BlockSpec((2, H, H), lambda i: (0, 0, 0)),
        ],
        out_specs=[
            pl.BlockSpec((block_n, H), lambda i: (i, 0)),
            pl.BlockSpec((block_n, H), lambda i: (i, 0)),
            pl.BlockSpec((block_n, H), lambda i: (i, 0)),
        ],
        out_shape=[
            jax.ShapeDtypeStruct((n, H), jnp.float32),
            jax.ShapeDtypeStruct((n, H), jnp.float32),
            jax.ShapeDtypeStruct((n, H), jnp.float32),
        ],
    )(xp, aemb, wab)


def _bwt_body(bemb_ref, wc_ref, out_ref):
    for l in range(wc_ref.shape[0]):
        for f in range(3):
            out_ref[l, f] = jnp.dot(bemb_ref[f], wc_ref[l],
                                    preferred_element_type=jnp.float32)


def _bwt(bemb, wc):
    l = wc.shape[0]
    return pl.pallas_call(
        _bwt_body,
        out_shape=jax.ShapeDtypeStruct((l, 3, 128, H), jnp.float32),
    )(bemb, wc)


def _edge_body(g_ref, eai_ref, bwt_ref, w2_ref, b1_ref, b2_ref,
               mlo_ref, mhi_ref):
    iota = lax.broadcasted_iota(jnp.int32, (1, 128), 1)
    z = g_ref[...] + b1_ref[...]
    for f in range(3):
        oh = (eai_ref[:, f][:, None] == iota).astype(jnp.float32)
        z = z + jnp.dot(oh, bwt_ref[f], preferred_element_type=jnp.float32)
    z = jnp.maximum(z, 0.0)
    m = jnp.dot(z, w2_ref[...],
                preferred_element_type=jnp.float32) + b2_ref[...]
    mlo_ref[...] = m[:, :128]
    mhi_ref[...] = m[:, 128:]


def _edge_mlp(g, eai, bwt_l, w2, b1, b2, block_e):
    e = g.shape[0]
    return pl.pallas_call(
        _edge_body,
        grid=(e // block_e,),
        in_specs=[
            pl.BlockSpec((block_e, H), lambda i: (i, 0)),
            pl.BlockSpec((block_e, 8), lambda i: (i, 0)),
            pl.BlockSpec((3, 128, H), lambda i: (0, 0, 0)),
            pl.BlockSpec((H, H), lambda i: (0, 0)),
            pl.BlockSpec((1, H), lambda i: (0, 0)),
            pl.BlockSpec((1, H), lambda i: (0, 0)),
        ],
        out_specs=[
            pl.BlockSpec((block_e, 128), lambda i: (i, 0)),
            pl.BlockSpec((block_e, 128), lambda i: (i, 0)),
        ],
        out_shape=[
            jax.ShapeDtypeStruct((e, 128), jnp.float32),
            jax.ShapeDtypeStruct((e, 128), jnp.float32),
        ],
    )(g, eai, bwt_l, w2, b1, b2)


def _post_body(h_ref, alo_ref, ahi_ref, dpk_ref,
               pw1_ref, pb1_ref, pw2_ref, pb2_ref, wab_ref,
               h_out_ref, hd_out_ref, hs_out_ref):
    h = h_ref[...]
    dc = dpk_ref[:, 0:1]
    logd = dpk_ref[:, 1:2]
    inv_logd = dpk_ref[:, 2:3]
    has_e = dpk_ref[:, 3:4]

    s1 = jnp.concatenate([alo_ref[0], ahi_ref[0]], axis=1)
    mxr = jnp.concatenate([alo_ref[1], ahi_ref[1]], axis=1)
    mnr = jnp.concatenate([alo_ref[2], ahi_ref[2]], axis=1)
    sq = jnp.concatenate([alo_ref[3], ahi_ref[3]], axis=1)

    mean = s1 / dc
    mx = jnp.where(has_e > 0.0, mxr, 0.0)
    mn = jnp.where(has_e > 0.0, mnr, 0.0)
    mean_sq = sq / dc
    std = jnp.sqrt(jnp.maximum(mean_sq - mean * mean, 0.0) + EPS)

    acc = jnp.dot(h, pw1_ref[0], preferred_element_type=jnp.float32)
    aggs = (mean, mx, mn, std)
    for a in range(4):
        agg = aggs[a]
        acc += jnp.dot(agg, pw1_ref[1 + 3 * a],
                       preferred_element_type=jnp.float32)
        acc += jnp.dot(agg * logd, pw1_ref[2 + 3 * a],
                       preferred_element_type=jnp.float32)
        acc += jnp.dot(agg * inv_logd, pw1_ref[3 + 3 * a],
                       preferred_element_type=jnp.float32)
    z = jnp.maximum(acc + pb1_ref[...], 0.0)
    h_new = jnp.dot(z, pw2_ref[...],
                    preferred_element_type=jnp.float32) + pb2_ref[...] + h
    h_out_ref[...] = h_new
    hd_out_ref[...] = jnp.dot(h_new, wab_ref[0],
                              preferred_element_type=jnp.float32)
    hs_out_ref[...] = jnp.dot(h_new, wab_ref[1],
                              preferred_element_type=jnp.float32)


def _post_mlp(h, alo, ahi, dpk, pw1, pb1, pw2, pb2, wab, block_n):
    n = h.shape[0]
    blk = lambda: pl.BlockSpec((block_n, H), lambda i: (i, 0))
    ablk = lambda: pl.BlockSpec((4, block_n, 128), lambda i: (0, i, 0))
    return pl.pallas_call(
        _post_body,
        grid=(n // block_n,),
        in_specs=[
            blk(), ablk(), ablk(),
            pl.BlockSpec((block_n, 128), lambda i: (i, 0)),
            pl.BlockSpec((13, H, H), lambda i: (0, 0, 0)),
            pl.BlockSpec((1, H), lambda i: (0, 0)),
            pl.BlockSpec((H, H), lambda i: (0, 0)),
            pl.BlockSpec((1, H), lambda i: (0, 0)),
            pl.BlockSpec((2, H, H), lambda i: (0, 0, 0)),
        ],
        out_specs=[blk(), blk(), blk()],
        out_shape=[
            jax.ShapeDtypeStruct((n, H), jnp.float32),
            jax.ShapeDtypeStruct((n, H), jnp.float32),
            jax.ShapeDtypeStruct((n, H), jnp.float32),
        ],
    )(h, alo, ahi, dpk, pw1, pb1, pw2, pb2, wab)


def kernel(atom_emb, bond_emb, pre_W1, pre_b1, pre_W2, pre_b2,
           post_W1, post_b1, post_W2, post_b2, x, edge_index, edge_attr):
    n = x.shape[0]
    e = edge_index.shape[1]
    l_total = pre_W1.shape[0]
    npad = NW * NPW

    src = edge_index[0]
    dst = edge_index[1]
    perm = jnp.argsort(dst)
    dst_s = dst[perm].astype(jnp.int32)
    src_s = src[perm].astype(jnp.int32)
    eai_s = jnp.zeros((e, 8), jnp.int32).at[:, :3].set(
        edge_attr[perm].astype(jnp.int32))
    rowptr = jnp.searchsorted(dst_s, jnp.arange(n + 1, dtype=jnp.int32),
                              side="left").astype(jnp.int32)
    ptrp = jnp.full((npad + 24,), e, jnp.int32).at[:n + 1].set(rowptr)

    deg = (rowptr[1:] - rowptr[:-1]).astype(jnp.float32)
    dc = jnp.maximum(deg, 1.0)
    logd = jnp.log(deg + 1.0)
    inv_logd = 1.0 / jnp.where(logd > 0, logd, 1.0)
    has_e = (deg > 0).astype(jnp.float32)
    dpk = jnp.zeros((n, 128), jnp.float32)
    dpk = dpk.at[:, 0].set(dc).at[:, 1].set(logd)
    dpk = dpk.at[:, 2].set(inv_logd).at[:, 3].set(has_e)

    w_ab = jnp.stack([pre_W1[:, 0:H, :], pre_W1[:, H:2 * H, :]],
                     axis=1)
    w_c = pre_W1[:, 2 * H:, :]
    aemb = jnp.zeros((9, 128, H), jnp.float32).at[:, :100, :].set(atom_emb)
    bemb = jnp.zeros((3, 128, H), jnp.float32).at[:, :100, :].set(bond_emb)
    xp = jnp.zeros((n, 16), jnp.int32).at[:, :9].set(x.astype(jnp.int32))

    bwt = _bwt(bemb, w_c)

    block_n = 1000 if n % 1000 == 0 else n
    block_e = 1280 if e % 1280 == 0 else e

    h, hd, hs = _encode(xp, aemb, w_ab[0], block_n)
    for l in range(l_total):
        g = _sc_gather(hd, hs, dst_s, src_s)
        mlo, mhi = _edge_mlp(g, eai_s, bwt[l], pre_W2[l], pre_b1[l][None],
                             pre_b2[l][None], block_e)
        alo = _sc_segreduce(mlo, ptrp, npad)[:, :n, :]
        ahi = _sc_segreduce(mhi, ptrp, npad)[:, :n, :]
        wab_next = w_ab[(l + 1) % l_total]
        h, hd, hs = _post_mlp(h, alo, ahi, dpk, post_W1[l].reshape(13, H, H),
                              post_b1[l][None], post_W2[l], post_b2[l][None],
                              wab_next, block_n)
    return h
